# baseline (speedup 1.0000x reference)
import functools

import jax
import jax.numpy as jnp
from jax import lax
from jax.experimental import pallas as pl
from jax.experimental.pallas import tpu as pltpu
from jax.experimental.pallas import tpu_sc as plsc

VOCAB = 100000
D = 64
B = 16384

NC = 2
NS = 16
L = 16
NW = NC * NS
BPW = B // NW
HALF = BPW // 2
NBLK = HALF // L
ROWS_PER_STEP = 16

TCB = 2048
TCG = -(-VOCAB // TCB)


def _transpose_body(xt_ref, out_ref):
    out_ref[...] = xt_ref[...].T


_tc_rowmajor = functools.partial(
    pl.pallas_call,
    out_shape=jax.ShapeDtypeStruct((VOCAB, D), jnp.float32),
    grid=(TCG,),
    in_specs=[pl.BlockSpec((D, TCB), lambda i: (0, i))],
    out_specs=pl.BlockSpec((TCB, D), lambda i: (i, 0)),
)(_transpose_body)


def _rsqrt16(x):
    i = lax.bitcast_convert_type(x, jnp.int32)
    i = jnp.int32(0x5F3759DF) - (i >> 1)
    y = lax.bitcast_convert_type(i, jnp.float32)
    half_x = x * 0.5
    for _ in range(3):
        y = y * (1.5 - half_x * y * y)
    return y


def _sc_body(center_hbm, context_hbm, ctab_hbm, xtab_hbm, out_hbm,
             cidx_v, xidx_v, crows_v, xrows_v, cout_v, sem):
    wid = lax.axis_index("s") * NC + lax.axis_index("c")
    base = wid * BPW

    pltpu.sync_copy(center_hbm.at[pl.ds(base, BPW)], cidx_v)
    pltpu.sync_copy(context_hbm.at[pl.ds(base, BPW)], xidx_v)

    lane = lax.iota(jnp.int32, L)

    for half in range(2):
        hbase = half * HALF

        def fetch(step, _):
            r0 = step * ROWS_PER_STEP
            cvec = cidx_v[pl.ds(hbase + r0, ROWS_PER_STEP)]
            xvec = xidx_v[pl.ds(hbase + r0, ROWS_PER_STEP)]
            for k in range(ROWS_PER_STEP):
                pltpu.async_copy(ctab_hbm.at[pl.ds(cvec[k], 1)],
                                 crows_v.at[pl.ds(r0 + k, 1)], sem)
                pltpu.async_copy(xtab_hbm.at[pl.ds(xvec[k], 1)],
                                 xrows_v.at[pl.ds(r0 + k, 1)], sem)
            return 0

        lax.fori_loop(0, HALF // ROWS_PER_STEP, fetch, 0)

        pltpu.make_async_copy(ctab_hbm.at[pl.ds(0, HALF)], crows_v, sem).wait()
        pltpu.make_async_copy(xtab_hbm.at[pl.ds(0, HALF)], xrows_v, sem).wait()

        def blk(b, _):
            rowv = lane + b * L
            dot = jnp.zeros((L,), jnp.float32)
            cc = jnp.zeros((L,), jnp.float32)
            xx = jnp.zeros((L,), jnp.float32)
            for d in range(D):
                colv = jnp.full((L,), d, jnp.int32)
                cv = plsc.load_gather(crows_v, [rowv, colv])
                xv = plsc.load_gather(xrows_v, [rowv, colv])
                dot = dot + cv * xv
                cc = cc + cv * cv
                xx = xx + xv * xv
            m = jnp.maximum(cc, 1e-12) * jnp.maximum(xx, 1e-12)
            prob = (1.0 + dot * _rsqrt16(m)) * 0.5
            cout_v[pl.ds(hbase + b * L, L)] = prob
            return 0

        lax.fori_loop(0, NBLK, blk, 0)

    pltpu.sync_copy(cout_v, out_hbm.at[pl.ds(base, BPW)])


_sc_call = functools.partial(
    pl.kernel,
    out_type=jax.ShapeDtypeStruct((B,), jnp.float32),
    mesh=plsc.VectorSubcoreMesh(core_axis_name="c", subcore_axis_name="s",
                                num_cores=NC, num_subcores=NS),
    compiler_params=pltpu.CompilerParams(needs_layout_passes=False,
                                         use_tc_tiling_on_sc=True),
    scratch_types=[
        pltpu.VMEM((BPW,), jnp.int32),
        pltpu.VMEM((BPW,), jnp.int32),
        pltpu.VMEM((HALF, D), jnp.float32),
        pltpu.VMEM((HALF, D), jnp.float32),
        pltpu.VMEM((BPW,), jnp.float32),
        pltpu.SemaphoreType.DMA,
    ],
)(_sc_body)


@jax.jit
def kernel(center, context, center_table, context_table):
    ctab = _tc_rowmajor(center_table.T)
    xtab = _tc_rowmajor(context_table.T)
    out = _sc_call(center, context, ctab, xtab)
    return out.reshape(B, 1)


# speedup vs baseline: 1.1512x; 1.1512x over previous
import functools

import jax
import jax.numpy as jnp
from jax import lax
from jax.experimental import pallas as pl
from jax.experimental.pallas import tpu as pltpu
from jax.experimental.pallas import tpu_sc as plsc

VOCAB = 100000
D = 64
B = 16384

NC = 2
NS = 16
L = 16
NW = NC * NS
BPW = B // NW
CHUNK = 128
NPASS = BPW // CHUNK
NBLK = CHUNK // L
RPS = 16


def _rsqrt16(x):
    i = lax.bitcast_convert_type(x, jnp.int32)
    i = jnp.int32(0x5F3759DF) - (i >> 1)
    y = lax.bitcast_convert_type(i, jnp.float32)
    half_x = x * 0.5
    for _ in range(3):
        y = y * (1.5 - half_x * y * y)
    return y


def _sc_body(center_hbm, context_hbm, ctab_hbm, xtab_hbm, out_hbm,
             cidx_v, xidx_v, crows0_v, xrows0_v, crows1_v, xrows1_v,
             cout_v, sem0, sem1):
    wid = lax.axis_index("s") * NC + lax.axis_index("c")
    base = wid * BPW

    pltpu.sync_copy(center_hbm.at[pl.ds(base, BPW)], cidx_v)
    pltpu.sync_copy(context_hbm.at[pl.ds(base, BPW)], xidx_v)

    lane = lax.iota(jnp.int32, L)
    bufs = ((crows0_v, xrows0_v, sem0), (crows1_v, xrows1_v, sem1))

    def fetch(p, par):
        crows_v, xrows_v, sem = bufs[par]
        for step in range(CHUNK // RPS):
            r0 = step * RPS
            cvec = cidx_v[pl.ds(p * CHUNK + r0, RPS)]
            xvec = xidx_v[pl.ds(p * CHUNK + r0, RPS)]
            for k in range(RPS):
                pltpu.async_copy(ctab_hbm.at[pl.ds(cvec[k], 1)],
                                 crows_v.at[pl.ds(r0 + k, 1)], sem)
                pltpu.async_copy(xtab_hbm.at[pl.ds(xvec[k], 1)],
                                 xrows_v.at[pl.ds(r0 + k, 1)], sem)

    def drain(par):
        crows_v, xrows_v, sem = bufs[par]
        pltpu.make_async_copy(ctab_hbm.at[pl.ds(0, CHUNK)], crows_v,
                              sem).wait()
        pltpu.make_async_copy(xtab_hbm.at[pl.ds(0, CHUNK)], xrows_v,
                              sem).wait()

    def compute(p, par):
        crows_v, xrows_v, _ = bufs[par]

        def blk(b, _):
            rowv = lane + b * L
            dot = jnp.zeros((L,), jnp.float32)
            cc = jnp.zeros((L,), jnp.float32)
            xx = jnp.zeros((L,), jnp.float32)
            for d in range(D):
                colv = jnp.full((L,), d, jnp.int32)
                cv = plsc.load_gather(crows_v, [rowv, colv])
                xv = plsc.load_gather(xrows_v, [rowv, colv])
                dot = dot + cv * xv
                cc = cc + cv * cv
                xx = xx + xv * xv
            m = jnp.maximum(cc, 1e-12) * jnp.maximum(xx, 1e-12)
            prob = (1.0 + dot * _rsqrt16(m)) * 0.5
            cout_v[pl.ds(p * CHUNK + b * L, L)] = prob
            return 0

        lax.fori_loop(0, NBLK, blk, 0)

    fetch(0, 0)
    for p in range(NPASS):
        drain(p % 2)
        if p + 1 < NPASS:
            fetch(p + 1, (p + 1) % 2)
        compute(p, p % 2)

    pltpu.sync_copy(cout_v, out_hbm.at[pl.ds(base, BPW)])


_sc_call = functools.partial(
    pl.kernel,
    out_type=jax.ShapeDtypeStruct((B,), jnp.float32),
    mesh=plsc.VectorSubcoreMesh(core_axis_name="c", subcore_axis_name="s",
                                num_cores=NC, num_subcores=NS),
    compiler_params=pltpu.CompilerParams(needs_layout_passes=False,
                                         use_tc_tiling_on_sc=True),
    scratch_types=[
        pltpu.VMEM((BPW,), jnp.int32),
        pltpu.VMEM((BPW,), jnp.int32),
        pltpu.VMEM((CHUNK, D), jnp.float32),
        pltpu.VMEM((CHUNK, D), jnp.float32),
        pltpu.VMEM((CHUNK, D), jnp.float32),
        pltpu.VMEM((CHUNK, D), jnp.float32),
        pltpu.VMEM((BPW,), jnp.float32),
        pltpu.SemaphoreType.DMA,
        pltpu.SemaphoreType.DMA,
    ],
)(_sc_body)


@jax.jit
def kernel(center, context, center_table, context_table):
    out = _sc_call(center, context, center_table, context_table)
    return out.reshape(B, 1)


# speedup vs baseline: 1.1536x; 1.0021x over previous
import functools

import jax
import jax.numpy as jnp
from jax import lax
from jax.experimental import pallas as pl
from jax.experimental.pallas import tpu as pltpu
from jax.experimental.pallas import tpu_sc as plsc

VOCAB = 100000
D = 64
B = 16384

NC = 2
NS = 16
L = 16
NW = NC * NS
BPW = B // NW
CHUNK = 128
NPASS = BPW // CHUNK
NBLK = CHUNK // L
RPS = 16


def _rsqrt16(x):
    i = lax.bitcast_convert_type(x, jnp.int32)
    i = jnp.int32(0x5F3759DF) - (i >> 1)
    y = lax.bitcast_convert_type(i, jnp.float32)
    half_x = x * 0.5
    for _ in range(3):
        y = y * (1.5 - half_x * y * y)
    return y


def _sc_body(center_hbm, context_hbm, ctab_hbm, xtab_hbm, out_hbm,
             cidx_v, xidx_v, crows0_v, xrows0_v, crows1_v, xrows1_v,
             cout_v, sem0, sem1):
    wid = lax.axis_index("s") * NC + lax.axis_index("c")
    base = wid * BPW

    pltpu.sync_copy(center_hbm.at[pl.ds(base, BPW)], cidx_v)
    pltpu.sync_copy(context_hbm.at[pl.ds(base, BPW)], xidx_v)

    lane = lax.iota(jnp.int32, L)
    bufs = ((crows0_v, xrows0_v, sem0), (crows1_v, xrows1_v, sem1))

    def fetch(p, par):
        crows_v, xrows_v, sem = bufs[par]
        for step in range(CHUNK // RPS):
            r0 = step * RPS
            cvec = cidx_v[pl.ds(p * CHUNK + r0, RPS)]
            xvec = xidx_v[pl.ds(p * CHUNK + r0, RPS)]
            for k in range(RPS):
                pltpu.async_copy(ctab_hbm.at[pl.ds(cvec[k], 1)],
                                 crows_v.at[pl.ds(r0 + k, 1)], sem)
                pltpu.async_copy(xtab_hbm.at[pl.ds(xvec[k], 1)],
                                 xrows_v.at[pl.ds(r0 + k, 1)], sem)

    def drain(par):
        crows_v, xrows_v, sem = bufs[par]
        pltpu.make_async_copy(ctab_hbm.at[pl.ds(0, CHUNK)], crows_v,
                              sem).wait()
        pltpu.make_async_copy(xtab_hbm.at[pl.ds(0, CHUNK)], xrows_v,
                              sem).wait()

    def compute(p, par):
        crows_v, xrows_v, _ = bufs[par]

        def blk(b, _):
            rowv = lane + b * L
            dot = jnp.zeros((L,), jnp.float32)
            cc = jnp.zeros((L,), jnp.float32)
            xx = jnp.zeros((L,), jnp.float32)
            for d in range(D):
                colv = jnp.full((L,), d, jnp.int32)
                cv = plsc.load_gather(crows_v, [rowv, colv])
                xv = plsc.load_gather(xrows_v, [rowv, colv])
                dot = dot + cv * xv
                cc = cc + cv * cv
                xx = xx + xv * xv
            m = jnp.maximum(cc, 1e-12) * jnp.maximum(xx, 1e-12)
            prob = (1.0 + dot * _rsqrt16(m)) * 0.5
            cout_v[pl.ds(p * CHUNK + b * L, L)] = prob
            return 0

        lax.fori_loop(0, NBLK, blk, 0)

    fetch(0, 0)
    for p in range(NPASS):
        drain(p % 2)
        if p + 1 < NPASS:
            fetch(p + 1, (p + 1) % 2)
        compute(p, p % 2)

    pltpu.sync_copy(cout_v, out_hbm.at[pl.ds(base, BPW)])


_sc_call = functools.partial(
    pl.kernel,
    out_type=jax.ShapeDtypeStruct((B,), jnp.float32),
    mesh=plsc.VectorSubcoreMesh(core_axis_name="c", subcore_axis_name="s",
                                num_cores=NC, num_subcores=NS),
    compiler_params=pltpu.CompilerParams(needs_layout_passes=False,
                                         use_tc_tiling_on_sc=True,
                                         skip_device_barrier=True,
                                         disable_bounds_checks=True,
                                         disable_semaphore_checks=True),
    scratch_types=[
        pltpu.VMEM((BPW,), jnp.int32),
        pltpu.VMEM((BPW,), jnp.int32),
        pltpu.VMEM((CHUNK, D), jnp.float32),
        pltpu.VMEM((CHUNK, D), jnp.float32),
        pltpu.VMEM((CHUNK, D), jnp.float32),
        pltpu.VMEM((CHUNK, D), jnp.float32),
        pltpu.VMEM((BPW,), jnp.float32),
        pltpu.SemaphoreType.DMA,
        pltpu.SemaphoreType.DMA,
    ],
)(_sc_body)


@jax.jit
def kernel(center, context, center_table, context_table):
    out = _sc_call(center, context, center_table, context_table)
    return out.reshape(B, 1)


# speedup vs baseline: 1.1874x; 1.0293x over previous
import functools

import jax
import jax.numpy as jnp
from jax import lax
from jax.experimental import pallas as pl
from jax.experimental.pallas import tpu as pltpu
from jax.experimental.pallas import tpu_sc as plsc

VOCAB = 100000
D = 64
B = 16384

NC = 2
NS = 16
L = 16
NW = NC * NS
BPW = B // NW
HALF = BPW // 2
NBLK = HALF // L
ROWS_PER_STEP = 16


def _rsqrt16(x):
    i = lax.bitcast_convert_type(x, jnp.int32)
    i = jnp.int32(0x5F3759DF) - (i >> 1)
    y = lax.bitcast_convert_type(i, jnp.float32)
    half_x = x * 0.5
    for _ in range(3):
        y = y * (1.5 - half_x * y * y)
    return y


def _sc_body(center_hbm, context_hbm, ctab_hbm, xtab_hbm, out_hbm,
             cidx_v, xidx_v, crows_v, xrows_v, cout_v, sem):
    wid = lax.axis_index("s") * NC + lax.axis_index("c")
    base = wid * BPW

    pltpu.sync_copy(center_hbm.at[pl.ds(base, BPW)], cidx_v)
    pltpu.sync_copy(context_hbm.at[pl.ds(base, BPW)], xidx_v)

    lane = lax.iota(jnp.int32, L)

    for half in range(2):
        hbase = half * HALF

        def fetch(step, _):
            r0 = step * ROWS_PER_STEP
            cvec = cidx_v[pl.ds(hbase + r0, ROWS_PER_STEP)]
            xvec = xidx_v[pl.ds(hbase + r0, ROWS_PER_STEP)]
            for k in range(ROWS_PER_STEP):
                pltpu.async_copy(ctab_hbm.at[pl.ds(cvec[k], 1)],
                                 crows_v.at[pl.ds(r0 + k, 1)], sem)
                pltpu.async_copy(xtab_hbm.at[pl.ds(xvec[k], 1)],
                                 xrows_v.at[pl.ds(r0 + k, 1)], sem)
            return 0

        lax.fori_loop(0, HALF // ROWS_PER_STEP, fetch, 0)

        pltpu.make_async_copy(ctab_hbm.at[pl.ds(0, HALF)], crows_v, sem).wait()
        pltpu.make_async_copy(xtab_hbm.at[pl.ds(0, HALF)], xrows_v, sem).wait()

        def blk(b, _):
            rowv = lane + b * L
            dot = jnp.zeros((L,), jnp.float32)
            cc = jnp.zeros((L,), jnp.float32)
            xx = jnp.zeros((L,), jnp.float32)
            for d in range(D):
                colv = jnp.full((L,), d, jnp.int32)
                cv = plsc.load_gather(crows_v, [rowv, colv])
                xv = plsc.load_gather(xrows_v, [rowv, colv])
                dot = dot + cv * xv
                cc = cc + cv * cv
                xx = xx + xv * xv
            m = jnp.maximum(cc, 1e-12) * jnp.maximum(xx, 1e-12)
            prob = (1.0 + dot * _rsqrt16(m)) * 0.5
            cout_v[pl.ds(hbase + b * L, L)] = prob
            return 0

        lax.fori_loop(0, NBLK, blk, 0)

    pltpu.sync_copy(cout_v, out_hbm.at[pl.ds(base, BPW)])


_sc_call = functools.partial(
    pl.kernel,
    out_type=jax.ShapeDtypeStruct((B,), jnp.float32),
    mesh=plsc.VectorSubcoreMesh(core_axis_name="c", subcore_axis_name="s",
                                num_cores=NC, num_subcores=NS),
    compiler_params=pltpu.CompilerParams(needs_layout_passes=False,
                                         use_tc_tiling_on_sc=True),
    scratch_types=[
        pltpu.VMEM((BPW,), jnp.int32),
        pltpu.VMEM((BPW,), jnp.int32),
        pltpu.VMEM((HALF, D), jnp.float32),
        pltpu.VMEM((HALF, D), jnp.float32),
        pltpu.VMEM((BPW,), jnp.float32),
        pltpu.SemaphoreType.DMA,
    ],
)(_sc_body)


@jax.jit
def kernel(center, context, center_table, context_table):
    out = _sc_call(center, context, center_table, context_table)
    return out.reshape(B, 1)


# speedup vs baseline: 1.2031x; 1.0132x over previous
import functools

import jax
import jax.numpy as jnp
from jax import lax
from jax.experimental import pallas as pl
from jax.experimental.pallas import tpu as pltpu
from jax.experimental.pallas import tpu_sc as plsc

VOCAB = 100000
D = 64
B = 16384

NC = 2
NS = 16
L = 16
NW = NC * NS
BPW = B // NW
HALF = BPW // 2
NBLK = HALF // L
ROWS_PER_STEP = 16


def _rsqrt16(x):
    i = lax.bitcast_convert_type(x, jnp.int32)
    i = jnp.int32(0x5F3759DF) - (i >> 1)
    y = lax.bitcast_convert_type(i, jnp.float32)
    half_x = x * 0.5
    for _ in range(3):
        y = y * (1.5 - half_x * y * y)
    return y


def _sc_body(center_hbm, context_hbm, ctab_hbm, xtab_hbm, out_hbm,
             cidx_v, xidx_v, crows_v, xrows_v, cout_v, sem):
    wid = lax.axis_index("s") * NC + lax.axis_index("c")
    base = wid * BPW

    pltpu.sync_copy(center_hbm.at[pl.ds(base, BPW)], cidx_v)
    pltpu.sync_copy(context_hbm.at[pl.ds(base, BPW)], xidx_v)

    lane = lax.iota(jnp.int32, L)

    for half in range(2):
        hbase = half * HALF

        def fetch(step, _):
            r0 = step * ROWS_PER_STEP
            cvec = cidx_v[pl.ds(hbase + r0, ROWS_PER_STEP)]
            xvec = xidx_v[pl.ds(hbase + r0, ROWS_PER_STEP)]
            for k in range(ROWS_PER_STEP):
                pltpu.async_copy(ctab_hbm.at[pl.ds(cvec[k], 1)],
                                 crows_v.at[pl.ds(r0 + k, 1)], sem)
                pltpu.async_copy(xtab_hbm.at[pl.ds(xvec[k], 1)],
                                 xrows_v.at[pl.ds(r0 + k, 1)], sem)
            return 0

        lax.fori_loop(0, HALF // ROWS_PER_STEP, fetch, 0)

        pltpu.make_async_copy(ctab_hbm.at[pl.ds(0, HALF)], crows_v, sem).wait()
        pltpu.make_async_copy(xtab_hbm.at[pl.ds(0, HALF)], xrows_v, sem).wait()

        def blk(b, _):
            rowv = lane + b * L
            acc = [jnp.zeros((L,), jnp.float32) for _ in range(6)]
            for d in range(D):
                colv = jnp.full((L,), d, jnp.int32)
                cv = plsc.load_gather(crows_v, [rowv, colv])
                xv = plsc.load_gather(xrows_v, [rowv, colv])
                o = 3 * (d & 1)
                acc[o] = acc[o] + cv * xv
                acc[o + 1] = acc[o + 1] + cv * cv
                acc[o + 2] = acc[o + 2] + xv * xv
            dot = acc[0] + acc[3]
            cc = acc[1] + acc[4]
            xx = acc[2] + acc[5]
            m = jnp.maximum(cc, 1e-12) * jnp.maximum(xx, 1e-12)
            prob = (1.0 + dot * _rsqrt16(m)) * 0.5
            cout_v[pl.ds(hbase + b * L, L)] = prob
            return 0

        lax.fori_loop(0, NBLK, blk, 0)

    pltpu.sync_copy(cout_v, out_hbm.at[pl.ds(base, BPW)])


_sc_call = functools.partial(
    pl.kernel,
    out_type=jax.ShapeDtypeStruct((B,), jnp.float32),
    mesh=plsc.VectorSubcoreMesh(core_axis_name="c", subcore_axis_name="s",
                                num_cores=NC, num_subcores=NS),
    compiler_params=pltpu.CompilerParams(needs_layout_passes=False,
                                         use_tc_tiling_on_sc=True),
    scratch_types=[
        pltpu.VMEM((BPW,), jnp.int32),
        pltpu.VMEM((BPW,), jnp.int32),
        pltpu.VMEM((HALF, D), jnp.float32),
        pltpu.VMEM((HALF, D), jnp.float32),
        pltpu.VMEM((BPW,), jnp.float32),
        pltpu.SemaphoreType.DMA,
    ],
)(_sc_body)


@jax.jit
def kernel(center, context, center_table, context_table):
    out = _sc_call(center, context, center_table, context_table)
    return out.reshape(B, 1)
